# Initial kernel scaffold; baseline (speedup 1.0000x reference)
import jax
import jax.numpy as jnp
from jax.experimental import pallas as pl


def kernel(x):
    raise NotImplementedError("write your pallas kernel here")


# speedup vs baseline: 1.7898x; 1.7898x over previous
import jax
import jax.numpy as jnp
from jax import lax
from jax.experimental import pallas as pl

B, C, D = 1024, 64, 1024
BB = 8


def _fused_body(x_ref, logits_ref, latent_ref):
    xb = x_ref[...]
    sq = jnp.sum(xb * xb, axis=1)
    logits_ref[...] = jnp.sqrt(sq)
    m = jnp.max(sq, axis=1, keepdims=True)
    d_iota = lax.broadcasted_iota(jnp.int32, (BB, D), 1)
    idx = jnp.min(jnp.where(sq == m, d_iota, jnp.int32(D)), axis=1)
    onehot = (d_iota == idx[:, None]).astype(jnp.float32)
    latent_ref[...] = jnp.sum(xb * onehot[:, None, :], axis=2)


@jax.jit
def kernel(x):
    logits, latent = pl.pallas_call(
        _fused_body,
        grid=(B // BB,),
        in_specs=[pl.BlockSpec((BB, C, D), lambda i: (i, 0, 0))],
        out_specs=[
            pl.BlockSpec((BB, D), lambda i: (i, 0)),
            pl.BlockSpec((BB, C), lambda i: (i, 0)),
        ],
        out_shape=[
            jax.ShapeDtypeStruct((B, D), jnp.float32),
            jax.ShapeDtypeStruct((B, C), jnp.float32),
        ],
    )(x)
    return (logits, latent)


# speedup vs baseline: 2.4489x; 1.3682x over previous
import jax
import jax.numpy as jnp
from jax import lax
from jax.experimental import pallas as pl

B, C, D = 1024, 64, 1024
BB = 16


def _fused_body(x_ref, logits_ref, latent_ref):
    xb = x_ref[...]
    sq = jnp.sum(xb * xb, axis=1)
    logits_ref[...] = jnp.sqrt(sq)
    m = jnp.max(sq, axis=1, keepdims=True)
    d_iota = lax.broadcasted_iota(jnp.int32, (BB, D), 1)
    idx = jnp.min(jnp.where(sq == m, d_iota, jnp.int32(D)), axis=1)
    onehot = (d_iota == idx[:, None]).astype(jnp.float32)
    latent_ref[...] = jnp.sum(xb * onehot[:, None, :], axis=2)


@jax.jit
def kernel(x):
    logits, latent = pl.pallas_call(
        _fused_body,
        grid=(B // BB,),
        in_specs=[pl.BlockSpec((BB, C, D), lambda i: (i, 0, 0))],
        out_specs=[
            pl.BlockSpec((BB, D), lambda i: (i, 0)),
            pl.BlockSpec((BB, C), lambda i: (i, 0)),
        ],
        out_shape=[
            jax.ShapeDtypeStruct((B, D), jnp.float32),
            jax.ShapeDtypeStruct((B, C), jnp.float32),
        ],
    )(x)
    return (logits, latent)


# speedup vs baseline: 2.8956x; 1.1824x over previous
import jax
import jax.numpy as jnp
from jax import lax
from jax.experimental import pallas as pl

B, C, D = 1024, 64, 1024
BB = 32


def _fused_body(x_ref, logits_ref, latent_ref):
    xb = x_ref[...]
    sq = jnp.sum(xb * xb, axis=1)
    logits_ref[...] = jnp.sqrt(sq)
    m = jnp.max(sq, axis=1, keepdims=True)
    d_iota = lax.broadcasted_iota(jnp.int32, (BB, D), 1)
    idx = jnp.min(jnp.where(sq == m, d_iota, jnp.int32(D)), axis=1)
    onehot = (d_iota == idx[:, None]).astype(jnp.float32)
    latent_ref[...] = jnp.sum(xb * onehot[:, None, :], axis=2)


@jax.jit
def kernel(x):
    logits, latent = pl.pallas_call(
        _fused_body,
        grid=(B // BB,),
        in_specs=[pl.BlockSpec((BB, C, D), lambda i: (i, 0, 0))],
        out_specs=[
            pl.BlockSpec((BB, D), lambda i: (i, 0)),
            pl.BlockSpec((BB, C), lambda i: (i, 0)),
        ],
        out_shape=[
            jax.ShapeDtypeStruct((B, D), jnp.float32),
            jax.ShapeDtypeStruct((B, C), jnp.float32),
        ],
    )(x)
    return (logits, latent)


# speedup vs baseline: 3.1490x; 1.0875x over previous
import jax
import jax.numpy as jnp
from jax import lax
from jax.experimental import pallas as pl

B, C, D = 1024, 64, 1024
BB = 64


def _fused_body(x_ref, logits_ref, latent_ref):
    xb = x_ref[...]
    sq = jnp.sum(xb * xb, axis=1)
    logits_ref[...] = jnp.sqrt(sq)
    m = jnp.max(sq, axis=1, keepdims=True)
    d_iota = lax.broadcasted_iota(jnp.int32, (BB, D), 1)
    idx = jnp.min(jnp.where(sq == m, d_iota, jnp.int32(D)), axis=1)
    onehot = (d_iota == idx[:, None]).astype(jnp.float32)
    latent_ref[...] = jnp.sum(xb * onehot[:, None, :], axis=2)


@jax.jit
def kernel(x):
    logits, latent = pl.pallas_call(
        _fused_body,
        grid=(B // BB,),
        in_specs=[pl.BlockSpec((BB, C, D), lambda i: (i, 0, 0))],
        out_specs=[
            pl.BlockSpec((BB, D), lambda i: (i, 0)),
            pl.BlockSpec((BB, C), lambda i: (i, 0)),
        ],
        out_shape=[
            jax.ShapeDtypeStruct((B, D), jnp.float32),
            jax.ShapeDtypeStruct((B, C), jnp.float32),
        ],
    )(x)
    return (logits, latent)
